# Initial kernel scaffold; baseline (speedup 1.0000x reference)
import functools

import jax
import jax.numpy as jnp
from jax import lax
from jax.experimental import pallas as pl
from jax.experimental.pallas import tpu as pltpu
from jax.experimental.pallas import tpu_sc as plsc

VOCAB = 18
FEAT = 64
TOTAL = 16384 * 50
NC, NS = 2, 16
NW = NC * NS
PER_W = TOTAL // NW
CHUNK = 128
N_CHUNKS = PER_W // CHUNK


def _body(w_hbm, p_hbm, c_hbm, op_hbm, oc_hbm, idx_v, rows_v, sem):
    wid = lax.axis_index("s") * NC + lax.axis_index("c")
    base = wid * PER_W

    def run(idx_hbm, out_hbm):
        def chunk(i, carry):
            off = base + i * CHUNK
            pltpu.sync_copy(idx_hbm.at[pl.ds(off, CHUNK)], idx_v)
            pltpu.async_copy(w_hbm.at[idx_v], rows_v, sem).wait()
            pltpu.sync_copy(rows_v, out_hbm.at[pl.ds(off, CHUNK)])
            return carry
        lax.fori_loop(0, N_CHUNKS, chunk, 0)

    run(p_hbm, op_hbm)
    run(c_hbm, oc_hbm)


@jax.jit
def _lookup(patch_flat, context_flat, weight):
    mesh = plsc.VectorSubcoreMesh(core_axis_name="c", subcore_axis_name="s")
    f = pl.kernel(
        _body,
        out_type=(
            jax.ShapeDtypeStruct((TOTAL, FEAT), jnp.float32),
            jax.ShapeDtypeStruct((TOTAL, FEAT), jnp.float32),
        ),
        mesh=mesh,
        scratch_types=[
            pltpu.VMEM((CHUNK,), jnp.int32),
            pltpu.VMEM((CHUNK, FEAT), jnp.float32),
            pltpu.SemaphoreType.DMA,
        ],
    )
    return f(weight, patch_flat, context_flat)


def kernel(patch, context, weight):
    b, h = patch.shape
    out_p, out_c = _lookup(patch.reshape(-1), context.reshape(-1), weight)
    return (out_p.reshape(b, h, FEAT), out_c.reshape(b, h, FEAT))


# speedup vs baseline: 1.6769x; 1.6769x over previous
import functools

import jax
import jax.numpy as jnp
from jax import lax
from jax.experimental import pallas as pl
from jax.experimental.pallas import tpu as pltpu
from jax.experimental.pallas import tpu_sc as plsc

VOCAB = 18
FEAT = 64
TOTAL = 16384 * 50
NC, NS = 2, 16
NW = NC * NS
PER_W = TOTAL // NW
CHUNK = 128
N_CHUNKS = PER_W // CHUNK


def _body(w_hbm, p_hbm, c_hbm, op_hbm, oc_hbm, idx_v, rows_v, sem):
    wid = lax.axis_index("s") * NC + lax.axis_index("c")
    base = wid * PER_W

    def run(idx_hbm, out_hbm):
        def chunk(i, carry):
            off = base + i * CHUNK
            pltpu.sync_copy(idx_hbm.at[pl.ds(off, CHUNK)], idx_v)
            pltpu.async_copy(w_hbm.at[idx_v], rows_v, sem).wait()
            pltpu.sync_copy(rows_v, out_hbm.at[pl.ds(off, CHUNK)])
            return carry
        lax.fori_loop(0, N_CHUNKS, chunk, 0)

    run(p_hbm, op_hbm)
    run(c_hbm, oc_hbm)


@jax.jit
def _lookup(patch_flat, context_flat, weight):
    mesh = plsc.VectorSubcoreMesh(core_axis_name="c", subcore_axis_name="s")
    f = pl.kernel(
        _body,
        out_type=(
            jax.ShapeDtypeStruct((TOTAL, FEAT), jnp.float32),
            jax.ShapeDtypeStruct((TOTAL, FEAT), jnp.float32),
        ),
        mesh=mesh,
        scratch_types=[
            pltpu.VMEM((CHUNK,), jnp.int32),
            pltpu.VMEM((CHUNK, FEAT), jnp.float32),
            pltpu.SemaphoreType.DMA,
        ],
        compiler_params=pltpu.CompilerParams(use_tc_tiling_on_sc=False),
    )
    return f(weight, patch_flat, context_flat)


def kernel(patch, context, weight):
    b, h = patch.shape
    out_p, out_c = _lookup(patch.reshape(-1), context.reshape(-1), weight)
    return (out_p.reshape(b, h, FEAT), out_c.reshape(b, h, FEAT))


# speedup vs baseline: 1.6892x; 1.0073x over previous
import functools

import jax
import jax.numpy as jnp
from jax import lax
from jax.experimental import pallas as pl
from jax.experimental.pallas import tpu as pltpu
from jax.experimental.pallas import tpu_sc as plsc

VOCAB = 18
FEAT = 64
TOTAL = 16384 * 50
NC, NS = 2, 16
NW = NC * NS
PER_W = TOTAL // NW
CHUNK = 128
NCH = PER_W // CHUNK
NBUF = 8
NGRP = NCH // NBUF


def _body(w_hbm, p_hbm, c_hbm, op_hbm, oc_hbm, idx_v, bufs, gsem, ssem):
    wid = lax.axis_index("s") * NC + lax.axis_index("c")
    row0 = wid * PER_W
    crow0 = wid * NCH

    def run(idx2d_hbm, out_hbm):
        pltpu.sync_copy(idx2d_hbm.at[pl.ds(crow0, NCH)], idx_v)

        def gather(i, b):
            pltpu.async_copy(w_hbm.at[idx_v.at[i]], bufs.at[b], gsem.at[b])

        def gather_wait(i, b):
            pltpu.make_async_copy(
                w_hbm.at[idx_v.at[i]], bufs.at[b], gsem.at[b]).wait()

        def scatter(i, b):
            pltpu.async_copy(
                bufs.at[b], out_hbm.at[pl.ds(row0 + i * CHUNK, CHUNK)],
                ssem.at[b])

        def scatter_wait(b):
            pltpu.make_async_copy(
                bufs.at[b], out_hbm.at[pl.ds(row0, CHUNK)], ssem.at[b]).wait()

        for b in range(NBUF):
            gather(b, b)

        def group(g, carry):
            for b in range(NBUF):
                i = g * NBUF + b
                gather_wait(i, b)
                scatter(i, b)
            for b in range(NBUF):
                scatter_wait(b)
                gather((g + 1) * NBUF + b, b)
            return carry

        lax.fori_loop(0, NGRP - 1, group, 0)

        for b in range(NBUF):
            i = (NGRP - 1) * NBUF + b
            gather_wait(i, b)
            scatter(i, b)
        for b in range(NBUF):
            scatter_wait(b)

    run(p_hbm, op_hbm)
    run(c_hbm, oc_hbm)


@jax.jit
def _lookup(patch2d, context2d, weight):
    mesh = plsc.VectorSubcoreMesh(core_axis_name="c", subcore_axis_name="s")
    f = pl.kernel(
        _body,
        out_type=(
            jax.ShapeDtypeStruct((TOTAL, FEAT), jnp.float32),
            jax.ShapeDtypeStruct((TOTAL, FEAT), jnp.float32),
        ),
        mesh=mesh,
        scratch_types=[
            pltpu.VMEM((NCH, CHUNK), jnp.int32),
            pltpu.VMEM((NBUF, CHUNK, FEAT), jnp.float32),
            pltpu.SemaphoreType.DMA((NBUF,)),
            pltpu.SemaphoreType.DMA((NBUF,)),
        ],
        compiler_params=pltpu.CompilerParams(use_tc_tiling_on_sc=False),
    )
    return f(weight, patch2d, context2d)


def kernel(patch, context, weight):
    b, h = patch.shape
    out_p, out_c = _lookup(
        patch.reshape(TOTAL // CHUNK, CHUNK),
        context.reshape(TOTAL // CHUNK, CHUNK),
        weight,
    )
    return (out_p.reshape(b, h, FEAT), out_c.reshape(b, h, FEAT))


# speedup vs baseline: 7.5923x; 4.4947x over previous
import functools

import jax
import jax.numpy as jnp
from jax import lax
from jax.experimental import pallas as pl
from jax.experimental.pallas import tpu as pltpu
from jax.experimental.pallas import tpu_sc as plsc

VOCAB = 18
FEAT = 64
TOTAL = 16384 * 50
NC, NS = 2, 16
NW = NC * NS
PER_W = TOTAL // NW
CHUNK = 128
NCH = PER_W // CHUNK
NBUF = 8
NGRP = NCH // NBUF


def _body(w_hbm, p_hbm, c_hbm, op_hbm, oc_hbm, idx_v, bufs, table_v, gsem, ssem):
    wid = lax.axis_index("s") * NC + lax.axis_index("c")
    row0 = wid * PER_W
    crow0 = wid * NCH

    pltpu.sync_copy(w_hbm, table_v)

    def run(idx2d_hbm, out_hbm):
        pltpu.sync_copy(idx2d_hbm.at[pl.ds(crow0, NCH)], idx_v)

        def gather(i, b):
            pltpu.async_copy(table_v.at[idx_v.at[i]], bufs.at[b], gsem.at[b])

        def gather_wait(i, b):
            pltpu.make_async_copy(
                table_v.at[idx_v.at[i]], bufs.at[b], gsem.at[b]).wait()

        def scatter(i, b):
            pltpu.async_copy(
                bufs.at[b], out_hbm.at[pl.ds(row0 + i * CHUNK, CHUNK)],
                ssem.at[b])

        def scatter_wait(b):
            pltpu.make_async_copy(
                bufs.at[b], out_hbm.at[pl.ds(row0, CHUNK)], ssem.at[b]).wait()

        for b in range(NBUF):
            gather(b, b)

        def group(g, carry):
            for b in range(NBUF):
                i = g * NBUF + b
                gather_wait(i, b)
                scatter(i, b)
            for b in range(NBUF):
                scatter_wait(b)
                gather((g + 1) * NBUF + b, b)
            return carry

        lax.fori_loop(0, NGRP - 1, group, 0)

        for b in range(NBUF):
            i = (NGRP - 1) * NBUF + b
            gather_wait(i, b)
            scatter(i, b)
        for b in range(NBUF):
            scatter_wait(b)

    run(p_hbm, op_hbm)
    run(c_hbm, oc_hbm)


@jax.jit
def _lookup(patch2d, context2d, weight):
    mesh = plsc.VectorSubcoreMesh(core_axis_name="c", subcore_axis_name="s")
    f = pl.kernel(
        _body,
        out_type=(
            jax.ShapeDtypeStruct((TOTAL, FEAT), jnp.float32),
            jax.ShapeDtypeStruct((TOTAL, FEAT), jnp.float32),
        ),
        mesh=mesh,
        scratch_types=[
            pltpu.VMEM((NCH, CHUNK), jnp.int32),
            pltpu.VMEM((NBUF, CHUNK, FEAT), jnp.float32),
            pltpu.VMEM_SHARED((VOCAB, FEAT), jnp.float32),
            pltpu.SemaphoreType.DMA((NBUF,)),
            pltpu.SemaphoreType.DMA((NBUF,)),
        ],
        compiler_params=pltpu.CompilerParams(use_tc_tiling_on_sc=False),
    )
    return f(weight, patch2d, context2d)


def kernel(patch, context, weight):
    b, h = patch.shape
    out_p, out_c = _lookup(
        patch.reshape(TOTAL // CHUNK, CHUNK),
        context.reshape(TOTAL // CHUNK, CHUNK),
        weight,
    )
    return (out_p.reshape(b, h, FEAT), out_c.reshape(b, h, FEAT))


# speedup vs baseline: 39.7083x; 5.2301x over previous
import functools

import jax
import jax.numpy as jnp
from jax import lax
from jax.experimental import pallas as pl
from jax.experimental.pallas import tpu as pltpu
from jax.experimental.pallas import tpu_sc as plsc

VOCAB = 18
FEAT = 64
BATCH = 16384
HIST = 50
NC, NS, L = 2, 16, 16
NW = NC * NS
NBT = BATCH // 128
BTW = NBT // NW
BW = BTW * 128
NFT = FEAT // 8


TSTRIDE = 80
TSZ = VOCAB * TSTRIDE


def _unit(idx_v, tsw, buf, h):
    def btg_body(btg, c0):
        def j_body(j, c1):
            i_vec = idx_v[h, pl.ds(btg * 128 + j * L, L)]
            ibase = i_vec * TSTRIDE + (i_vec & 15)

            @plsc.parallel_loop(0, FEAT, unroll=8)
            def f_loop(f):
                v = plsc.load_gather(tsw, [ibase + f])
                buf[f >> 3, btg, f & 7, pl.ds(j * L, L)] = v

            return c1
        return lax.fori_loop(0, 8, j_body, c0)
    lax.fori_loop(0, BTW, btg_body, 0)


def _body(w_hbm, p_hbm, c_hbm, op_hbm, oc_hbm, idx_v, table_v, tsw, bufs,
          ssem):
    wid = lax.axis_index("s") * NC + lax.axis_index("c")
    bt0 = wid * BTW
    col0 = wid * BW

    pltpu.sync_copy(w_hbm, table_v)
    iota = lax.iota(jnp.int32, L)
    for r in range(VOCAB):
        base = r * TSTRIDE + (r & 15)
        for k in range(FEAT // L):
            plsc.store_scatter(
                tsw, [iota + (base + k * L)], table_v[r, pl.ds(k * L, L)])

    def scatter(out_hbm, h, slot):
        pltpu.async_copy(
            bufs.at[slot], out_hbm.at[h, :, pl.ds(bt0, BTW)], ssem.at[slot])

    def scatter_wait(out_hbm, slot):
        pltpu.make_async_copy(
            bufs.at[slot], out_hbm.at[0, :, pl.ds(bt0, BTW)],
            ssem.at[slot]).wait()

    def run(idxT_hbm, out_hbm):
        pltpu.sync_copy(idxT_hbm.at[:, pl.ds(col0, BW)], idx_v)
        for slot in range(2):
            _unit(idx_v, tsw, bufs.at[slot], slot)
            scatter(out_hbm, slot, slot)

        def pair(hh, carry):
            for slot in range(2):
                h = hh * 2 + slot
                scatter_wait(out_hbm, slot)
                _unit(idx_v, tsw, bufs.at[slot], h)
                scatter(out_hbm, h, slot)
            return carry

        lax.fori_loop(1, HIST // 2, pair, 0)
        for slot in range(2):
            scatter_wait(out_hbm, slot)

    run(p_hbm, op_hbm)
    run(c_hbm, oc_hbm)


@jax.jit
def _lookup(patch_t, context_t, weight):
    mesh = plsc.VectorSubcoreMesh(core_axis_name="c", subcore_axis_name="s")
    out5 = jax.ShapeDtypeStruct((HIST, NFT, NBT, 8, 128), jnp.float32)
    f = pl.kernel(
        _body,
        out_type=(out5, out5),
        mesh=mesh,
        scratch_types=[
            pltpu.VMEM((HIST, BW), jnp.int32),
            pltpu.VMEM((VOCAB, FEAT), jnp.float32),
            pltpu.VMEM((TSZ,), jnp.float32),
            pltpu.VMEM((2, NFT, BTW, 8, 128), jnp.float32),
            pltpu.SemaphoreType.DMA((2,)),
        ],
        compiler_params=pltpu.CompilerParams(
            use_tc_tiling_on_sc=False, needs_layout_passes=False),
    )
    return f(weight, patch_t, context_t)


def kernel(patch, context, weight):
    out5_p, out5_c = _lookup(patch.T, context.T, weight)

    def relabel(o5):
        return o5.transpose((2, 4, 0, 1, 3)).reshape(BATCH, HIST, FEAT)

    return (relabel(out5_p), relabel(out5_c))
